# Initial kernel scaffold; baseline (speedup 1.0000x reference)
import jax
import jax.numpy as jnp
from jax.experimental import pallas as pl


def kernel(input, pe_table):
    raise NotImplementedError("write your pallas kernel here")


# speedup vs baseline: 2.9974x; 2.9974x over previous
import jax
import jax.numpy as jnp
from jax.experimental import pallas as pl

_BLOCK = 1024


def _gather_rows_kernel(pe_ref, out_ref):
    out_ref[...] = pe_ref[...]


def kernel(input, pe_table):
    length = input.shape[1]
    max_len, d = pe_table.shape
    grid = pl.cdiv(length, _BLOCK)
    out = pl.pallas_call(
        _gather_rows_kernel,
        grid=(grid,),
        in_specs=[pl.BlockSpec((_BLOCK, d), lambda i: (i, 0))],
        out_specs=pl.BlockSpec((_BLOCK, d), lambda i: (i, 0)),
        out_shape=jax.ShapeDtypeStruct((length, d), pe_table.dtype),
    )(pe_table)
    return out[None]
